# Initial kernel scaffold; baseline (speedup 1.0000x reference)
import jax
import jax.numpy as jnp
from jax.experimental import pallas as pl


def kernel(emb, instance, kernel, training_mask):
    raise NotImplementedError("write your pallas kernel here")


# speedup vs baseline: 67.1573x; 67.1573x over previous
import functools

import jax
import jax.numpy as jnp
from jax import lax
from jax.experimental import pallas as pl
from jax.experimental.pallas import tpu as pltpu

C = 4
MAXL = 8
DELTA_V = 0.5
DELTA_D = 1.5
ROWS = 256
LANES = 1024
R_CHUNK = 64
NCH = ROWS // R_CHUNK


def _pass1_body(emb_ref, inst_ref, ker_ref, tm_ref, out_ref, acc_ref):
    j = pl.program_id(1)

    @pl.when(j == 0)
    def _():
        acc_ref[...] = jnp.zeros_like(acc_ref)

    e = emb_ref[0]
    inst = inst_ref[0]
    ker = ker_ref[0]
    tm = tm_ref[0]
    inst_m = jnp.where(tm > 0.5, inst, 0.0)
    ik = jnp.where(ker > 0.5, inst_m, 0.0)
    for l in range(MAXL):
        mk = (ik == float(l)).astype(jnp.float32)
        mi = (inst_m == float(l)).astype(jnp.float32)
        acc_ref[l, 0] += jnp.sum(mk, axis=0)
        acc_ref[l, 1] += jnp.sum(mi, axis=0)
        for c in range(C):
            acc_ref[l, 2 + c] += jnp.sum(mk * e[c], axis=0)

    @pl.when(j == NCH - 1)
    def _():
        out_ref[0] = jnp.sum(acc_ref[...], axis=2)


def _pass1_tc(emb_t, inst, ker, tm):
    B = inst.shape[0]
    grid = (B, NCH)
    return pl.pallas_call(
        _pass1_body,
        grid=grid,
        in_specs=[
            pl.BlockSpec((1, C, R_CHUNK, LANES), lambda b, j: (b, 0, j, 0)),
            pl.BlockSpec((1, R_CHUNK, LANES), lambda b, j: (b, j, 0)),
            pl.BlockSpec((1, R_CHUNK, LANES), lambda b, j: (b, j, 0)),
            pl.BlockSpec((1, R_CHUNK, LANES), lambda b, j: (b, j, 0)),
        ],
        out_specs=pl.BlockSpec((1, MAXL, 6), lambda b, j: (b, 0, 0)),
        out_shape=jax.ShapeDtypeStruct((B, MAXL, 6), jnp.float32),
        scratch_shapes=[pltpu.VMEM((MAXL, 6, LANES), jnp.float32)],
        compiler_params=pltpu.CompilerParams(
            dimension_semantics=("arbitrary", "arbitrary")),
    )(emb_t, inst, ker, tm)


def _finalize(counts_k, cnt_i, sum_v, emb_mean, msq):
    lbl = lax.broadcasted_iota(jnp.int32, (1, MAXL), 1)
    present = counts_k > 0.0
    num_instance = jnp.sum(present.astype(jnp.float32))
    per_lbl = sum_v / jnp.maximum(cnt_i, 1.0)
    nz = jnp.logical_and(present, lbl != 0)
    first_nz = jnp.min(jnp.where(nz, lbl, MAXL))
    agg_mask = jnp.logical_and(nz, lbl != first_nz)
    n_agg = jnp.sum(agg_mask.astype(jnp.float32))
    l_agg = jnp.sum(jnp.where(agg_mask, per_lbl, 0.0)) / jnp.maximum(n_agg, 1.0)

    lr = lax.broadcasted_iota(jnp.int32, (MAXL, MAXL), 0)
    lc = lax.broadcasted_iota(jnp.int32, (MAXL, MAXL), 1)
    pres_r = jnp.broadcast_to(present.reshape(MAXL, 1), (MAXL, MAXL))
    pres_c = jnp.broadcast_to(present.reshape(1, MAXL), (MAXL, MAXL))
    pair_mask = pres_r & pres_c & (lr != lc) & (lr != 0) & (lc != 0)
    g = jnp.dot(emb_mean, emb_mean.T, preferred_element_type=jnp.float32)
    pd2 = msq.reshape(MAXL, 1) + msq.reshape(1, MAXL) - 2.0 * g
    pd2 = jnp.where(pair_mask, jnp.maximum(pd2, 0.0), float(C))
    pd = jnp.sqrt(pd2)
    pdm = jnp.maximum(2.0 * DELTA_D - pd, 0.0) ** 2
    ldv = jnp.log(pdm + 1.0)
    n_pair = jnp.sum(pair_mask.astype(jnp.float32))
    l_dis = jnp.sum(jnp.where(pair_mask, ldv, 0.0)) / jnp.maximum(n_pair, 1.0)
    l_dis = jnp.where(num_instance > 2.0, l_dis, 0.0)

    reg_mask = jnp.logical_and(present, lbl != 0)
    rv = jnp.log(jnp.sqrt(jnp.where(reg_mask, msq, 1.0)) + 1.0)
    l_reg = jnp.sum(jnp.where(reg_mask, rv, 0.0)) / jnp.maximum(
        num_instance, 1.0) * 0.001
    loss = l_agg + l_dis + l_reg
    return jnp.where(num_instance > 1.0, loss, 0.0)


def _pass2_body(stats_ref, emb_ref, inst_ref, tm_ref, out_ref, acc_ref):
    j = pl.program_id(1)

    @pl.when(j == 0)
    def _():
        acc_ref[...] = jnp.zeros_like(acc_ref)

    stats = stats_ref[0]
    counts_k = stats[:, 0].reshape(1, MAXL)
    sums = stats[:, 2:2 + C]
    emb_mean = sums / jnp.maximum(counts_k, 1.0).reshape(MAXL, 1)
    zero_row = (lax.broadcasted_iota(jnp.int32, (MAXL, C), 0) == 0)
    emb_mean = jnp.where(zero_row, 0.0, emb_mean)
    msq = jnp.sum(emb_mean * emb_mean, axis=1).reshape(1, MAXL)

    e = emb_ref[0]
    inst = inst_ref[0]
    tm = tm_ref[0]
    inst_m = jnp.where(tm > 0.5, inst, 0.0)
    esq = e[0] * e[0] + e[1] * e[1] + e[2] * e[2] + e[3] * e[3]
    sel_dot = jnp.zeros_like(inst)
    sel_msq = jnp.zeros_like(inst)
    ohs = []
    for l in range(MAXL):
        oh = (inst_m == float(l)).astype(jnp.float32)
        ohs.append(oh)
        dot = (e[0] * emb_mean[l, 0] + e[1] * emb_mean[l, 1]
               + e[2] * emb_mean[l, 2] + e[3] * emb_mean[l, 3])
        sel_dot += oh * dot
        sel_msq += oh * msq[0, l]
    d2 = jnp.maximum(esq - 2.0 * sel_dot + sel_msq, 0.0)
    d = jnp.sqrt(d2)
    t = jnp.maximum(d - DELTA_V, 0.0)
    v = jnp.log(t * t + 1.0)
    for l in range(MAXL):
        acc_ref[l] += jnp.sum(ohs[l] * v, axis=0)

    @pl.when(j == NCH - 1)
    def _():
        sum_v = jnp.sum(acc_ref[...], axis=1).reshape(1, MAXL)
        cnt_i = stats[:, 1].reshape(1, MAXL)
        loss = _finalize(counts_k, cnt_i, sum_v, emb_mean, msq)
        out_ref[0, 0, :] = jnp.full((128,), loss, jnp.float32)


def _pass2_tc(stats, emb_t, inst, tm):
    B = inst.shape[0]
    grid = (B, NCH)
    return pl.pallas_call(
        _pass2_body,
        grid=grid,
        in_specs=[
            pl.BlockSpec((1, MAXL, 6), lambda b, j: (b, 0, 0)),
            pl.BlockSpec((1, C, R_CHUNK, LANES), lambda b, j: (b, 0, j, 0)),
            pl.BlockSpec((1, R_CHUNK, LANES), lambda b, j: (b, j, 0)),
            pl.BlockSpec((1, R_CHUNK, LANES), lambda b, j: (b, j, 0)),
        ],
        out_specs=pl.BlockSpec((1, 1, 128), lambda b, j: (b, 0, 0)),
        out_shape=jax.ShapeDtypeStruct((B, 1, 128), jnp.float32),
        scratch_shapes=[pltpu.VMEM((MAXL, LANES), jnp.float32)],
        compiler_params=pltpu.CompilerParams(
            dimension_semantics=("arbitrary", "arbitrary")),
    )(stats, emb_t, inst, tm)


def kernel(emb, instance, kernel, training_mask):
    B, H, W, _ = emb.shape
    P = H * W
    emb_t = jnp.transpose(emb.reshape(B, P, C), (0, 2, 1))
    emb_t = emb_t.reshape(B, C, ROWS, LANES)
    inst = instance.reshape(B, ROWS, LANES)
    ker = kernel.reshape(B, ROWS, LANES)
    tm = training_mask.reshape(B, ROWS, LANES)
    stats = _pass1_tc(emb_t, inst, ker, tm)
    losses = _pass2_tc(stats, emb_t, inst, tm)
    return jnp.mean(losses[:, 0, 0])
